# baseline (speedup 1.0000x reference)
import functools

import jax
import jax.numpy as jnp
from jax import lax
from jax.experimental import pallas as pl
from jax.experimental.pallas import tpu as pltpu
from jax.experimental.pallas import tpu_sc as plsc

_LANES = 16


@functools.lru_cache(maxsize=None)
def _make_sc_embed(B, D, NC, NS, CH, NBUF):
    NW = NC * NS
    b_per_w = B // NW
    n_chunks = b_per_w // CH
    nbuf = min(NBUF, n_chunks)
    mesh = plsc.VectorSubcoreMesh(core_axis_name="c", subcore_axis_name="s")

    @functools.partial(
        pl.kernel,
        mesh=mesh,
        out_type=jax.ShapeDtypeStruct((B, D), jnp.float32),
        scratch_types=(
            [pltpu.VMEM((b_per_w,), jnp.int32),
             pltpu.VMEM((D,), jnp.int32)]
            + [pltpu.VMEM((CH, D), jnp.float32)] * nbuf
            + [pltpu.SemaphoreType.DMA] * (2 * nbuf)
        ),
    )
    def k(ids_hbm, table_hbm, pos_hbm, out_hbm, idx_v, pos_v, *rest):
        bufs = rest[:nbuf]
        gsem = rest[nbuf:2 * nbuf]
        osem = rest[2 * nbuf:]
        wid = lax.axis_index("s") * NC + lax.axis_index("c")
        base = wid * b_per_w
        pltpu.sync_copy(ids_hbm.at[pl.ds(base, b_per_w)], idx_v)
        pltpu.sync_copy(pos_hbm.at[0], pos_v)

        def add_pos(buf):
            def col_body(v, _):
                sl = pl.ds(v * _LANES, _LANES)
                pv = pos_v[sl].astype(jnp.float32)

                def row_body(r, _):
                    buf[r, sl] = buf[r, sl] + pv
                    return 0

                lax.fori_loop(0, CH, row_body, 0, unroll=4)
                return 0

            lax.fori_loop(0, D // _LANES, col_body, 0)

        gcp = [None] * nbuf
        ocp = [None] * nbuf
        for c in range(min(nbuf - 1, n_chunks)):
            gcp[c] = pltpu.async_copy(
                table_hbm.at[idx_v.at[pl.ds(c * CH, CH)]], bufs[c], gsem[c])
        for c in range(n_chunks):
            s = c % nbuf
            gcp[s].wait()
            nxt = c + nbuf - 1
            if nbuf > 1 and nxt < n_chunks:
                sp = nxt % nbuf
                if ocp[sp] is not None:
                    ocp[sp].wait()
                gcp[sp] = pltpu.async_copy(
                    table_hbm.at[idx_v.at[pl.ds(nxt * CH, CH)]],
                    bufs[sp], gsem[sp])
            add_pos(bufs[s])
            ocp[s] = pltpu.async_copy(
                bufs[s], out_hbm.at[pl.ds(base + c * CH, CH)], osem[s])
        for s in range(nbuf):
            if ocp[s] is not None:
                ocp[s].wait()

    return k


def kernel(input_ids, embeddings, positional_id):
    B = input_ids.shape[0]
    D = embeddings.shape[1]
    info = plsc.get_sparse_core_info()
    NC, NS = info.num_cores, info.num_subcores
    if input_ids.dtype != jnp.int32:
        input_ids = input_ids.astype(jnp.int32)
    k = _make_sc_embed(B, D, NC, NS, 8, 4)
    return k(input_ids, embeddings, positional_id)


# speedup vs baseline: 1.0248x; 1.0248x over previous
import functools

import jax
import jax.numpy as jnp
from jax import lax
from jax.experimental import pallas as pl
from jax.experimental.pallas import tpu as pltpu
from jax.experimental.pallas import tpu_sc as plsc

_LANES = 16


@functools.lru_cache(maxsize=None)
def _make_sc_embed(B, D, NC, NS, CH, NBUF):
    NW = NC * NS
    b_per_w = B // NW
    n_chunks = b_per_w // CH
    nbuf = min(NBUF, n_chunks)
    mesh = plsc.VectorSubcoreMesh(core_axis_name="c", subcore_axis_name="s")

    @functools.partial(
        pl.kernel,
        mesh=mesh,
        out_type=jax.ShapeDtypeStruct((B, D), jnp.float32),
        scratch_types=(
            [pltpu.VMEM((b_per_w,), jnp.int32),
             pltpu.VMEM((D,), jnp.int32)]
            + [pltpu.VMEM((CH, D), jnp.float32)] * nbuf
            + [pltpu.SemaphoreType.DMA] * (2 * nbuf)
        ),
    )
    def k(ids_hbm, table_hbm, pos_hbm, out_hbm, idx_v, pos_v, *rest):
        bufs = rest[:nbuf]
        gsem = rest[nbuf:2 * nbuf]
        osem = rest[2 * nbuf:]
        wid = lax.axis_index("s") * NC + lax.axis_index("c")
        base = wid * b_per_w
        pltpu.sync_copy(ids_hbm.at[pl.ds(base, b_per_w)], idx_v)
        pltpu.sync_copy(pos_hbm.at[0], pos_v)

        def add_pos(buf, r0, r1):
            def col_body(v, _):
                sl = pl.ds(v * _LANES, _LANES)
                pv = pos_v[sl].astype(jnp.float32)

                def row_body(r, _):
                    buf[r, sl] = buf[r, sl] + pv
                    return 0

                lax.fori_loop(r0, r1, row_body, 0, unroll=4)
                return 0

            lax.fori_loop(0, D // _LANES, col_body, 0)

        gcp = [None] * nbuf
        ocp = [None] * nbuf
        for c in range(min(nbuf - 1, n_chunks)):
            gcp[c] = pltpu.async_copy(
                table_hbm.at[idx_v.at[pl.ds(c * CH, CH)]], bufs[c], gsem[c])
        for c in range(n_chunks):
            s = c % nbuf
            gcp[s].wait()
            nxt = c + nbuf - 1
            if nbuf > 1 and nxt < n_chunks:
                sp = nxt % nbuf
                if ocp[sp] is not None:
                    for cp in ocp[sp]:
                        cp.wait()
                gcp[sp] = pltpu.async_copy(
                    table_hbm.at[idx_v.at[pl.ds(nxt * CH, CH)]],
                    bufs[sp], gsem[sp])
            H = CH // 2
            add_pos(bufs[s], 0, H)
            cpa = pltpu.async_copy(
                bufs[s].at[pl.ds(0, H)],
                out_hbm.at[pl.ds(base + c * CH, H)], osem[s])
            add_pos(bufs[s], H, CH)
            cpb = pltpu.async_copy(
                bufs[s].at[pl.ds(H, H)],
                out_hbm.at[pl.ds(base + c * CH + H, H)], osem[s])
            ocp[s] = (cpa, cpb)
        for s in range(nbuf):
            if ocp[s] is not None:
                for cp in ocp[s]:
                    cp.wait()

    return k


def kernel(input_ids, embeddings, positional_id):
    B = input_ids.shape[0]
    D = embeddings.shape[1]
    info = plsc.get_sparse_core_info()
    NC, NS = info.num_cores, info.num_subcores
    if input_ids.dtype != jnp.int32:
        input_ids = input_ids.astype(jnp.int32)
    k = _make_sc_embed(B, D, NC, NS, 16, 3)
    return k(input_ids, embeddings, positional_id)


# speedup vs baseline: 1.0366x; 1.0116x over previous
import functools

import jax
import jax.numpy as jnp
from jax import lax
from jax.experimental import pallas as pl
from jax.experimental.pallas import tpu as pltpu
from jax.experimental.pallas import tpu_sc as plsc

_LANES = 16


@functools.lru_cache(maxsize=None)
def _make_sc_embed(B, D, NC, NS, CH, NBUF):
    NW = NC * NS
    b_per_w = B // NW
    sizes = [CH] * (b_per_w // CH - 1) + [CH // 2, CH // 2]
    offs = []
    o = 0
    for sz in sizes:
        offs.append(o)
        o += sz
    n_chunks = len(sizes)
    nbuf = min(NBUF, n_chunks)
    mesh = plsc.VectorSubcoreMesh(core_axis_name="c", subcore_axis_name="s")

    @functools.partial(
        pl.kernel,
        mesh=mesh,
        out_type=jax.ShapeDtypeStruct((B, D), jnp.float32),
        scratch_types=(
            [pltpu.VMEM((b_per_w,), jnp.int32),
             pltpu.VMEM((D,), jnp.int32)]
            + [pltpu.VMEM((CH, D), jnp.float32)] * nbuf
            + [pltpu.SemaphoreType.DMA] * (2 * nbuf)
        ),
    )
    def k(ids_hbm, table_hbm, pos_hbm, out_hbm, idx_v, pos_v, *rest):
        bufs = rest[:nbuf]
        gsem = rest[nbuf:2 * nbuf]
        osem = rest[2 * nbuf:]
        wid = lax.axis_index("s") * NC + lax.axis_index("c")
        base = wid * b_per_w
        pltpu.sync_copy(ids_hbm.at[pl.ds(base, b_per_w)], idx_v)
        pltpu.sync_copy(pos_hbm.at[0], pos_v)

        def add_pos(buf, nrows):
            def col_body(v, _):
                sl = pl.ds(v * _LANES, _LANES)
                pv = pos_v[sl].astype(jnp.float32)

                def row_body(r, _):
                    buf[r, sl] = buf[r, sl] + pv
                    return 0

                lax.fori_loop(0, nrows, row_body, 0, unroll=4)
                return 0

            lax.fori_loop(0, D // _LANES, col_body, 0)

        def start_gather(c, s):
            return pltpu.async_copy(
                table_hbm.at[idx_v.at[pl.ds(offs[c], sizes[c])]],
                bufs[s].at[pl.ds(0, sizes[c])], gsem[s])

        gcp = [None] * nbuf
        ocp = [None] * nbuf
        for c in range(min(nbuf - 1, n_chunks)):
            gcp[c] = start_gather(c, c)
        for c in range(n_chunks):
            s = c % nbuf
            gcp[s].wait()
            nxt = c + nbuf - 1
            if nbuf > 1 and nxt < n_chunks:
                sp = nxt % nbuf
                if ocp[sp] is not None:
                    ocp[sp].wait()
                gcp[sp] = start_gather(nxt, sp)
            add_pos(bufs[s], sizes[c])
            ocp[s] = pltpu.async_copy(
                bufs[s].at[pl.ds(0, sizes[c])],
                out_hbm.at[pl.ds(base + offs[c], sizes[c])], osem[s])
        for s in range(nbuf):
            if ocp[s] is not None:
                ocp[s].wait()

    return k


def kernel(input_ids, embeddings, positional_id):
    B = input_ids.shape[0]
    D = embeddings.shape[1]
    info = plsc.get_sparse_core_info()
    NC, NS = info.num_cores, info.num_subcores
    if input_ids.dtype != jnp.int32:
        input_ids = input_ids.astype(jnp.int32)
    k = _make_sc_embed(B, D, NC, NS, 16, 3)
    return k(input_ids, embeddings, positional_id)


# speedup vs baseline: 1.0773x; 1.0392x over previous
import functools

import jax
import jax.numpy as jnp
from jax import lax
from jax.experimental import pallas as pl
from jax.experimental.pallas import tpu as pltpu
from jax.experimental.pallas import tpu_sc as plsc

_LANES = 16


@functools.lru_cache(maxsize=None)
def _make_sc_embed(B, D, NC, NS, CH, NBUF):
    NW = NC * NS
    b_per_w = B // NW
    n_chunks = b_per_w // CH
    nbuf = min(NBUF, n_chunks)
    mesh = plsc.VectorSubcoreMesh(core_axis_name="c", subcore_axis_name="s")

    @functools.partial(
        pl.kernel,
        mesh=mesh,
        out_type=jax.ShapeDtypeStruct((B, D), jnp.float32),
        scratch_types=(
            [pltpu.VMEM((b_per_w,), jnp.int32),
             pltpu.VMEM((D,), jnp.int32)]
            + [pltpu.VMEM((CH, D), jnp.float32)] * nbuf
            + [pltpu.SemaphoreType.DMA] * (2 * nbuf)
        ),
    )
    def k(ids_hbm, table_hbm, pos_hbm, out_hbm, idx_v, pos_v, *rest):
        bufs = rest[:nbuf]
        gsem = rest[nbuf:2 * nbuf]
        osem = rest[2 * nbuf:]
        wid = lax.axis_index("s") * NC + lax.axis_index("c")
        base = wid * b_per_w
        pltpu.sync_copy(ids_hbm.at[pl.ds(base, b_per_w)], idx_v)
        pltpu.sync_copy(pos_hbm.at[0], pos_v)

        def add_pos(buf):
            def col_body(v, _):
                sl = pl.ds(v * _LANES, _LANES)
                pv = pos_v[sl].astype(jnp.float32)

                def row_body(r, _):
                    buf[r, sl] = buf[r, sl] + pv
                    return 0

                lax.fori_loop(0, CH, row_body, 0, unroll=8)
                return 0

            lax.fori_loop(0, D // _LANES, col_body, 0)

        gcp = [None] * nbuf
        ocp = [None] * nbuf
        for c in range(min(nbuf - 1, n_chunks)):
            gcp[c] = pltpu.async_copy(
                table_hbm.at[idx_v.at[pl.ds(c * CH, CH)]], bufs[c], gsem[c])
        for c in range(n_chunks):
            s = c % nbuf
            gcp[s].wait()
            nxt = c + nbuf - 1
            if nbuf > 1 and nxt < n_chunks:
                sp = nxt % nbuf
                if ocp[sp] is not None:
                    ocp[sp].wait()
                gcp[sp] = pltpu.async_copy(
                    table_hbm.at[idx_v.at[pl.ds(nxt * CH, CH)]],
                    bufs[sp], gsem[sp])
            add_pos(bufs[s])
            ocp[s] = pltpu.async_copy(
                bufs[s], out_hbm.at[pl.ds(base + c * CH, CH)], osem[s])
        for s in range(nbuf):
            if ocp[s] is not None:
                ocp[s].wait()

    return k


def kernel(input_ids, embeddings, positional_id):
    B = input_ids.shape[0]
    D = embeddings.shape[1]
    info = plsc.get_sparse_core_info()
    NC, NS = info.num_cores, info.num_subcores
    if input_ids.dtype != jnp.int32:
        input_ids = input_ids.astype(jnp.int32)
    k = _make_sc_embed(B, D, NC, NS, 16, 3)
    return k(input_ids, embeddings, positional_id)
